# baseline (speedup 1.0000x reference)
import jax
import jax.numpy as jnp
from jax.experimental import pallas as pl
from jax.experimental.pallas import tpu as pltpu

_NUM_CODEBOOKS = 4
_K = 1024
_D = 64
_BETA = 0.25
_BLK = 1024


def _vq_kernel(x_ref, cb_ref, cb3_ref, csq_ref,
               xq_ref, ind_ref, loss_ref, unused_ref,
               counts_ref):
    step = pl.program_id(0)
    nsteps = pl.num_programs(0)

    @pl.when(step == 0)
    def _init():
        loss_ref[0, 0] = jnp.float32(0.0)
        counts_ref[...] = jnp.zeros_like(counts_ref)

    r = jnp.transpose(x_ref[...])
    xq = jnp.zeros_like(r)
    iota = jax.lax.broadcasted_iota(jnp.int32, (_K, _BLK), 0)
    ones8 = jnp.ones((8, _BLK), jnp.bfloat16)
    loss = jnp.float32(0.0)
    for level in range(_NUM_CODEBOOKS):
        a = r * r
        acc = a[0:8]
        for j in range(1, 8):
            acc = acc + a[8 * j:8 * j + 8]
        for w in (4, 2, 1):
            acc = acc[:w] + acc[w:2 * w]
        rsq = acc
        mm = jax.lax.dot_general(
            cb_ref[level], r, (((1,), (0,)), ((), ())),
            preferred_element_type=jnp.float32)
        dist = rsq - 2.0 * mm + csq_ref[level]
        minval = jnp.min(dist, axis=0, keepdims=True)
        idx = jnp.min(jnp.where(dist == minval, iota, _K),
                      axis=0).astype(jnp.int32)
        ohb = (iota == idx[None, :]).astype(jnp.bfloat16)
        e = jax.lax.dot_general(
            cb3_ref[level], ohb, (((0,), (0,)), ((), ())),
            preferred_element_type=jnp.float32)
        q = (e[0:_D] + e[_D:2 * _D]) + e[2 * _D:3 * _D]
        diff = q - r
        loss = loss + jnp.sum(diff * diff)
        hist = jax.lax.dot_general(
            ones8, ohb, (((1,), (1,)), ((), ())),
            preferred_element_type=jnp.float32)
        counts_ref[level] = counts_ref[level] + hist[0:1]
        ind_ref[0, level:level + 1, :] = idx[None, :]
        xq = xq + q
        r = r - q
    xq_ref[...] = jnp.transpose(xq)
    loss_ref[0, 0] += loss

    @pl.when(step == nsteps - 1)
    def _finalize():
        unused_ref[0, 0] = jnp.sum(
            (counts_ref[...] == 0.0).astype(jnp.int32))
        n_total = nsteps * _BLK
        loss_ref[0, 0] = loss_ref[0, 0] * jnp.float32(
            (1.0 + _BETA) / (n_total * _D))


def kernel(x, codebooks):
    orig_shape = x.shape
    latent = x.reshape(-1, _D)
    n = latent.shape[0]
    nblk = n // _BLK
    assert nblk * _BLK == n
    csq = jnp.sum(codebooks ** 2, axis=2)[:, :, None]

    bits = jax.lax.bitcast_convert_type(codebooks, jnp.uint32)
    hi = jax.lax.bitcast_convert_type(bits & jnp.uint32(0xFFFF0000),
                                      jnp.float32)
    rem = codebooks - hi
    rbits = jax.lax.bitcast_convert_type(rem, jnp.uint32)
    mid = jax.lax.bitcast_convert_type(rbits & jnp.uint32(0xFFFF0000),
                                       jnp.float32)
    lo = rem - mid
    cb3 = jnp.concatenate([hi, mid, lo],
                          axis=-1).astype(jnp.bfloat16)

    xq, ind, loss, unused = pl.pallas_call(
        _vq_kernel,
        grid=(nblk,),
        in_specs=[
            pl.BlockSpec((_BLK, _D), lambda i: (i, 0)),
            pl.BlockSpec((_NUM_CODEBOOKS, _K, _D), lambda i: (0, 0, 0)),
            pl.BlockSpec((_NUM_CODEBOOKS, _K, 3 * _D),
                         lambda i: (0, 0, 0)),
            pl.BlockSpec((_NUM_CODEBOOKS, _K, 1), lambda i: (0, 0, 0)),
        ],
        out_specs=[
            pl.BlockSpec((_BLK, _D), lambda i: (i, 0)),
            pl.BlockSpec((1, _NUM_CODEBOOKS, _BLK), lambda i: (i, 0, 0)),
            pl.BlockSpec(block_shape=(1, 1), index_map=lambda i: (0, 0),
                         memory_space=pltpu.SMEM),
            pl.BlockSpec(block_shape=(1, 1), index_map=lambda i: (0, 0),
                         memory_space=pltpu.SMEM),
        ],
        out_shape=[
            jax.ShapeDtypeStruct((n, _D), jnp.float32),
            jax.ShapeDtypeStruct((nblk, _NUM_CODEBOOKS, _BLK), jnp.int32),
            jax.ShapeDtypeStruct((1, 1), jnp.float32),
            jax.ShapeDtypeStruct((1, 1), jnp.int32),
        ],
        scratch_shapes=[pltpu.VMEM((_NUM_CODEBOOKS, 1, _K), jnp.float32)],
        compiler_params=pltpu.CompilerParams(
            dimension_semantics=("arbitrary",)),
    )(latent, codebooks, cb3, csq)

    x_q = xq.reshape(orig_shape)
    embed_inds = ind.transpose(1, 0, 2).reshape(
        _NUM_CODEBOOKS, *orig_shape[:-1])
    return (x_q, loss[0, 0], unused[0, 0], embed_inds)


# speedup vs baseline: 1.0353x; 1.0353x over previous
import jax
import jax.numpy as jnp
from jax.experimental import pallas as pl
from jax.experimental.pallas import tpu as pltpu

_NUM_CODEBOOKS = 4
_K = 1024
_D = 64
_BETA = 0.25
_BLK = 2048


def _vq_kernel(x_ref, cb_ref, cb3_ref, csq_ref,
               xq_ref, ind_ref, loss_ref, unused_ref,
               counts_ref):
    step = pl.program_id(0)
    nsteps = pl.num_programs(0)

    @pl.when(step == 0)
    def _init():
        loss_ref[0, 0] = jnp.float32(0.0)
        counts_ref[...] = jnp.zeros_like(counts_ref)

    r = jnp.transpose(x_ref[...])
    xq = jnp.zeros_like(r)
    iota = jax.lax.broadcasted_iota(jnp.int32, (_K, _BLK), 0)
    ones8 = jnp.ones((8, _BLK), jnp.bfloat16)
    loss = jnp.float32(0.0)
    for level in range(_NUM_CODEBOOKS):
        a = r * r
        acc = a[0:8]
        for j in range(1, 8):
            acc = acc + a[8 * j:8 * j + 8]
        for w in (4, 2, 1):
            acc = acc[:w] + acc[w:2 * w]
        rsq = acc
        mm = jax.lax.dot_general(
            cb_ref[level], r, (((1,), (0,)), ((), ())),
            preferred_element_type=jnp.float32)
        dist = rsq - 2.0 * mm + csq_ref[level]
        minval = jnp.min(dist, axis=0, keepdims=True)
        idx = jnp.min(jnp.where(dist == minval, iota, _K),
                      axis=0).astype(jnp.int32)
        ohb = (iota == idx[None, :]).astype(jnp.bfloat16)
        e = jax.lax.dot_general(
            cb3_ref[level], ohb, (((0,), (0,)), ((), ())),
            preferred_element_type=jnp.float32)
        q = (e[0:_D] + e[_D:2 * _D]) + e[2 * _D:3 * _D]
        diff = q - r
        loss = loss + jnp.sum(diff * diff)
        hist = jax.lax.dot_general(
            ones8, ohb, (((1,), (1,)), ((), ())),
            preferred_element_type=jnp.float32)
        counts_ref[level] = counts_ref[level] + hist[0:1]
        ind_ref[0, level:level + 1, :] = idx[None, :]
        xq = xq + q
        r = r - q
    xq_ref[...] = jnp.transpose(xq)
    loss_ref[0, 0] += loss

    @pl.when(step == nsteps - 1)
    def _finalize():
        unused_ref[0, 0] = jnp.sum(
            (counts_ref[...] == 0.0).astype(jnp.int32))
        n_total = nsteps * _BLK
        loss_ref[0, 0] = loss_ref[0, 0] * jnp.float32(
            (1.0 + _BETA) / (n_total * _D))


def kernel(x, codebooks):
    orig_shape = x.shape
    latent = x.reshape(-1, _D)
    n = latent.shape[0]
    nblk = n // _BLK
    assert nblk * _BLK == n
    csq = jnp.sum(codebooks ** 2, axis=2)[:, :, None]

    bits = jax.lax.bitcast_convert_type(codebooks, jnp.uint32)
    hi = jax.lax.bitcast_convert_type(bits & jnp.uint32(0xFFFF0000),
                                      jnp.float32)
    rem = codebooks - hi
    rbits = jax.lax.bitcast_convert_type(rem, jnp.uint32)
    mid = jax.lax.bitcast_convert_type(rbits & jnp.uint32(0xFFFF0000),
                                       jnp.float32)
    lo = rem - mid
    cb3 = jnp.concatenate([hi, mid, lo],
                          axis=-1).astype(jnp.bfloat16)

    xq, ind, loss, unused = pl.pallas_call(
        _vq_kernel,
        grid=(nblk,),
        in_specs=[
            pl.BlockSpec((_BLK, _D), lambda i: (i, 0)),
            pl.BlockSpec((_NUM_CODEBOOKS, _K, _D), lambda i: (0, 0, 0)),
            pl.BlockSpec((_NUM_CODEBOOKS, _K, 3 * _D),
                         lambda i: (0, 0, 0)),
            pl.BlockSpec((_NUM_CODEBOOKS, _K, 1), lambda i: (0, 0, 0)),
        ],
        out_specs=[
            pl.BlockSpec((_BLK, _D), lambda i: (i, 0)),
            pl.BlockSpec((1, _NUM_CODEBOOKS, _BLK), lambda i: (i, 0, 0)),
            pl.BlockSpec(block_shape=(1, 1), index_map=lambda i: (0, 0),
                         memory_space=pltpu.SMEM),
            pl.BlockSpec(block_shape=(1, 1), index_map=lambda i: (0, 0),
                         memory_space=pltpu.SMEM),
        ],
        out_shape=[
            jax.ShapeDtypeStruct((n, _D), jnp.float32),
            jax.ShapeDtypeStruct((nblk, _NUM_CODEBOOKS, _BLK), jnp.int32),
            jax.ShapeDtypeStruct((1, 1), jnp.float32),
            jax.ShapeDtypeStruct((1, 1), jnp.int32),
        ],
        scratch_shapes=[pltpu.VMEM((_NUM_CODEBOOKS, 1, _K), jnp.float32)],
        compiler_params=pltpu.CompilerParams(
            dimension_semantics=("arbitrary",)),
    )(latent, codebooks, cb3, csq)

    x_q = xq.reshape(orig_shape)
    embed_inds = ind.transpose(1, 0, 2).reshape(
        _NUM_CODEBOOKS, *orig_shape[:-1])
    return (x_q, loss[0, 0], unused[0, 0], embed_inds)
